# baseline (speedup 1.0000x reference)
import functools

import jax
import jax.numpy as jnp
from jax import lax
from jax.experimental import pallas as pl
from jax.experimental.pallas import tpu as pltpu
from jax.experimental.pallas import tpu_sc as plsc

EPS = 1e-5
K = 128
NC = 2
NS = 16
NW = NC * NS


def _sc_degree(nw, nchunk, npad, stripe):
    mesh = plsc.VectorSubcoreMesh(core_axis_name="c", subcore_axis_name="s")

    @functools.partial(
        pl.kernel,
        out_type=jax.ShapeDtypeStruct((NC, npad), jnp.float32),
        mesh=mesh,
        scratch_types=[
            pltpu.VMEM((nchunk, K), jnp.int32),
            pltpu.VMEM((K,), jnp.float32),
            pltpu.VMEM((stripe,), jnp.float32),
            pltpu.VMEM_SHARED((npad,), jnp.float32),
            pltpu.SemaphoreType.DMA,
        ],
    )
    def deg_kernel(dst_hbm, out_hbm, dstv, onesv, zv, deg_sh, ssem):
        c = lax.axis_index("c")
        s = lax.axis_index("s")
        wid = s * NC + c
        pltpu.sync_copy(dst_hbm.at[wid], dstv)
        ones16 = jnp.full((16,), 1.0, jnp.float32)
        zero16 = jnp.zeros((16,), jnp.float32)
        for i in range(K // 16):
            onesv[pl.ds(i * 16, 16)] = ones16

        def zbody(i, carry):
            zv[pl.ds(i * 16, 16)] = zero16
            return carry

        lax.fori_loop(0, stripe // 16, zbody, 0)
        pltpu.sync_copy(zv, deg_sh.at[pl.ds(s * stripe, stripe)])
        plsc.subcore_barrier()

        rd = 8

        def body(j, carry):
            @pl.when(j >= rd)
            def _():
                pltpu.make_async_copy(onesv, deg_sh.at[dstv.at[0]],
                                      ssem).wait()

            pltpu.async_copy(onesv, deg_sh.at[dstv.at[j]], ssem, add=True)
            return carry

        lax.fori_loop(0, nchunk, body, 0)

        def drain(j, carry):
            pltpu.make_async_copy(onesv, deg_sh.at[dstv.at[0]], ssem).wait()
            return carry

        lax.fori_loop(0, min(rd, nchunk), drain, 0)
        plsc.subcore_barrier()
        pltpu.sync_copy(deg_sh.at[pl.ds(s * stripe, stripe)],
                        out_hbm.at[c, pl.ds(s * stripe, stripe)])

    return deg_kernel


def _sc_aggregate(nw, nchunk, npad, stripe, d):
    mesh = plsc.VectorSubcoreMesh(core_axis_name="c", subcore_axis_name="s")

    nbuf = 2
    G = 8
    ngrp = nchunk // G
    assert nchunk % G == 0 and ngrp > 2 and nbuf <= G

    @functools.partial(
        pl.kernel,
        out_type=jax.ShapeDtypeStruct((NC, npad, d), jnp.float32),
        mesh=mesh,
        scratch_types=[
            pltpu.VMEM((2, G, K), jnp.int32),
            pltpu.VMEM((2, G, K), jnp.int32),
            pltpu.VMEM((nbuf, K, d), jnp.float32),
            pltpu.VMEM_SHARED((npad, d), jnp.float32),
            pltpu.SemaphoreType.DMA((nbuf,)),
            pltpu.SemaphoreType.DMA((2,)),
        ],
    )
    def agg_kernel(scaled_hbm, src_hbm, dst_hbm, out_hbm,
                   srcg, dstg, rows, acc_sh, gsems, isems):
        c = lax.axis_index("c")
        s = lax.axis_index("s")
        wid = s * NC + c
        zero16 = jnp.zeros((16,), jnp.float32)

        def zbody(i, carry):
            for j in range(d // 16):
                rows[0, i, pl.ds(j * 16, 16)] = zero16
            return carry

        lax.fori_loop(0, K, zbody, 0)
        for r in range(-(-stripe // K)):
            blk = min(K, stripe - r * K)
            pltpu.sync_copy(rows.at[0, pl.ds(0, blk)],
                            acc_sh.at[pl.ds(s * stripe + r * K, blk)])
        plsc.subcore_barrier()

        def start_grp(g, slot):
            pltpu.async_copy(src_hbm.at[wid, pl.ds(g * G, G)], srcg.at[slot],
                             isems.at[slot])
            pltpu.async_copy(dst_hbm.at[wid, pl.ds(g * G, G)], dstg.at[slot],
                             isems.at[slot])

        def wait_grp(slot):
            pltpu.make_async_copy(src_hbm.at[wid, pl.ds(0, G)], srcg.at[slot],
                                  isems.at[slot]).wait()
            pltpu.make_async_copy(dst_hbm.at[wid, pl.ds(0, G)], dstg.at[slot],
                                  isems.at[slot]).wait()

        start_grp(0, 0)
        start_grp(1, 1)
        wait_grp(0)
        for i in range(nbuf):
            pltpu.async_copy(scaled_hbm.at[srcg.at[0, i]], rows.at[i],
                             gsems.at[i])

        def body(j, carry):
            rslot = lax.rem(j, nbuf)
            gj = j // G
            pltpu.make_async_copy(scaled_hbm.at[srcg.at[0, 0]],
                                  rows.at[rslot], gsems.at[rslot]).wait()
            pltpu.sync_copy(rows.at[rslot],
                            acc_sh.at[dstg.at[lax.rem(gj, 2), lax.rem(j, G)]],
                            add=True)

            @pl.when((lax.rem(j, G) == 0) & (j > 0) & (gj + 1 < ngrp))
            def _():
                start_grp(gj + 1, lax.rem(gj + 1, 2))

            t = j + nbuf

            @pl.when(t < nchunk)
            def _():
                tg = t // G

                @pl.when(lax.rem(t, G) == 0)
                def _():
                    wait_grp(lax.rem(tg, 2))

                pltpu.async_copy(
                    scaled_hbm.at[srcg.at[lax.rem(tg, 2), lax.rem(t, G)]],
                    rows.at[rslot], gsems.at[rslot])

            return carry

        lax.fori_loop(0, nchunk, body, 0)
        plsc.subcore_barrier()
        pltpu.sync_copy(acc_sh.at[pl.ds(s * stripe, stripe)],
                        out_hbm.at[c, pl.ds(s * stripe, stripe)])

    return agg_kernel


def _tc_scale(dp3, x, w, npad):
    n, _ = x.shape
    d = w.shape[1]

    def body(dp_ref, x_ref, w_ref, scaled_ref, dinv_ref):
        deg = dp_ref[0] + dp_ref[1] + 1.0
        dinv = lax.rsqrt(deg)
        dinv_n = dinv[:n]
        h = jnp.dot(x_ref[...], w_ref[...], preferred_element_type=jnp.float32)
        scaled_ref[:n] = h * dinv_n
        scaled_ref[n:] = jnp.zeros((npad - n, d), jnp.float32)
        dinv_ref[...] = dinv_n

    return pl.pallas_call(
        body,
        out_shape=[
            jax.ShapeDtypeStruct((npad, d), jnp.float32),
            jax.ShapeDtypeStruct((n, 1), jnp.float32),
        ],
    )(dp3, x, w)


def _tc_finish(partials, scaled, dinv, x, b2, gamma2, beta2):
    n, d = x.shape

    def body(p_ref, scaled_ref, dinv_ref, x_ref, b_ref, g_ref, be_ref, out_ref):
        agg = p_ref[0, :n] + p_ref[1, :n] + scaled_ref[:n]
        conv = agg * dinv_ref[...] + b_ref[...]
        mean = jnp.mean(conv, axis=0, keepdims=True)
        cm = conv - mean
        var = jnp.mean(cm * cm, axis=0, keepdims=True)
        hbn = g_ref[...] * cm * lax.rsqrt(var + EPS) + be_ref[...]
        out_ref[...] = jnp.maximum(jnp.maximum(hbn, 0.0) + x_ref[...], 0.0)

    return pl.pallas_call(
        body,
        out_shape=jax.ShapeDtypeStruct((n, d), jnp.float32),
    )(partials, scaled, dinv, x, b2, gamma2, beta2)


def kernel(x, edge_index, W, b, gamma, beta):
    n, d_in = x.shape
    d = W.shape[1]
    e = edge_index.shape[1]

    npad_deg = (n // 256 + 1) * 256
    npad = (n // 128 + 1) * 128
    stripe = npad // NS
    per_w = -(-e // NW)
    nchunk = -(-(-(-per_w // K)) // 8) * 8
    epad = NW * nchunk * K
    pad_cnt = epad - e

    pad_idx = n + (jnp.arange(pad_cnt, dtype=jnp.int32) % (npad - n))
    src = jnp.concatenate([edge_index[0], pad_idx]).reshape(NW, nchunk, K)
    dst = jnp.concatenate([edge_index[1], pad_idx]).reshape(NW, nchunk, K)

    degp = _sc_degree(NW, nchunk, npad_deg, npad_deg // NS)(dst)
    scaled, dinv = _tc_scale(degp.reshape(NC, npad_deg, 1), x, W, npad)
    partials = _sc_aggregate(NW, nchunk, npad, stripe, d)(scaled, src, dst)
    return _tc_finish(partials, scaled, dinv, x,
                      b.reshape(1, d), gamma.reshape(1, d), beta.reshape(1, d))


# speedup vs baseline: 1.0724x; 1.0724x over previous
import functools

import jax
import jax.numpy as jnp
from jax import lax
from jax.experimental import pallas as pl
from jax.experimental.pallas import tpu as pltpu
from jax.experimental.pallas import tpu_sc as plsc

EPS = 1e-5
K = 64
NC = 2
NS = 16
NW = NC * NS


def _sc_degree(nw, nchunk, npad, stripe):
    mesh = plsc.VectorSubcoreMesh(core_axis_name="c", subcore_axis_name="s")

    @functools.partial(
        pl.kernel,
        out_type=jax.ShapeDtypeStruct((NC, npad), jnp.float32),
        mesh=mesh,
        scratch_types=[
            pltpu.VMEM((nchunk, K), jnp.int32),
            pltpu.VMEM((K,), jnp.float32),
            pltpu.VMEM((stripe,), jnp.float32),
            pltpu.VMEM_SHARED((npad,), jnp.float32),
            pltpu.SemaphoreType.DMA,
        ],
    )
    def deg_kernel(dst_hbm, out_hbm, dstv, onesv, zv, deg_sh, ssem):
        c = lax.axis_index("c")
        s = lax.axis_index("s")
        wid = s * NC + c
        pltpu.sync_copy(dst_hbm.at[wid], dstv)
        ones16 = jnp.full((16,), 1.0, jnp.float32)
        zero16 = jnp.zeros((16,), jnp.float32)
        for i in range(K // 16):
            onesv[pl.ds(i * 16, 16)] = ones16

        def zbody(i, carry):
            zv[pl.ds(i * 16, 16)] = zero16
            return carry

        lax.fori_loop(0, stripe // 16, zbody, 0)
        pltpu.sync_copy(zv, deg_sh.at[pl.ds(s * stripe, stripe)])
        plsc.subcore_barrier()

        rd = 8

        def body(j, carry):
            @pl.when(j >= rd)
            def _():
                pltpu.make_async_copy(onesv, deg_sh.at[dstv.at[0]],
                                      ssem).wait()

            pltpu.async_copy(onesv, deg_sh.at[dstv.at[j]], ssem, add=True)
            return carry

        lax.fori_loop(0, nchunk, body, 0)

        def drain(j, carry):
            pltpu.make_async_copy(onesv, deg_sh.at[dstv.at[0]], ssem).wait()
            return carry

        lax.fori_loop(0, min(rd, nchunk), drain, 0)
        plsc.subcore_barrier()
        pltpu.sync_copy(deg_sh.at[pl.ds(s * stripe, stripe)],
                        out_hbm.at[c, pl.ds(s * stripe, stripe)])

    return deg_kernel


def _sc_aggregate(nw, nchunk, npad, stripe, d):
    mesh = plsc.VectorSubcoreMesh(core_axis_name="c", subcore_axis_name="s")

    nbuf = 5
    G = 8
    ngrp = nchunk // G
    assert nchunk % G == 0 and ngrp > 2 and nbuf <= G

    @functools.partial(
        pl.kernel,
        out_type=jax.ShapeDtypeStruct((NC, npad, d), jnp.float32),
        mesh=mesh,
        scratch_types=[
            pltpu.VMEM((2, G, K), jnp.int32),
            pltpu.VMEM((2, G, K), jnp.int32),
            pltpu.VMEM((nbuf, K, d), jnp.float32),
            pltpu.VMEM_SHARED((npad, d), jnp.float32),
            pltpu.SemaphoreType.DMA((nbuf,)),
            pltpu.SemaphoreType.DMA((2,)),
        ],
    )
    def agg_kernel(scaled_hbm, src_hbm, dst_hbm, out_hbm,
                   srcg, dstg, rows, acc_sh, gsems, isems):
        c = lax.axis_index("c")
        s = lax.axis_index("s")
        wid = s * NC + c

        def start_grp(g, slot):
            pltpu.async_copy(src_hbm.at[wid, pl.ds(g * G, G)], srcg.at[slot],
                             isems.at[slot])
            pltpu.async_copy(dst_hbm.at[wid, pl.ds(g * G, G)], dstg.at[slot],
                             isems.at[slot])

        def wait_grp(slot):
            pltpu.make_async_copy(src_hbm.at[wid, pl.ds(0, G)], srcg.at[slot],
                                  isems.at[slot]).wait()
            pltpu.make_async_copy(dst_hbm.at[wid, pl.ds(0, G)], dstg.at[slot],
                                  isems.at[slot]).wait()

        start_grp(0, 0)
        start_grp(1, 1)
        zero16 = jnp.zeros((16,), jnp.float32)

        def zbody(i, carry):
            for j in range(d // 16):
                rows[0, i, pl.ds(j * 16, 16)] = zero16
            return carry

        lax.fori_loop(0, K, zbody, 0)
        for r in range(-(-stripe // K)):
            blk = min(K, stripe - r * K)
            pltpu.sync_copy(rows.at[0, pl.ds(0, blk)],
                            acc_sh.at[pl.ds(s * stripe + r * K, blk)])
        plsc.subcore_barrier()
        wait_grp(0)
        for i in range(nbuf):
            pltpu.async_copy(scaled_hbm.at[srcg.at[0, i]], rows.at[i],
                             gsems.at[i])

        def body(j, carry):
            rslot = lax.rem(j, nbuf)
            gj = j // G
            pltpu.make_async_copy(scaled_hbm.at[srcg.at[0, 0]],
                                  rows.at[rslot], gsems.at[rslot]).wait()
            pltpu.sync_copy(rows.at[rslot],
                            acc_sh.at[dstg.at[lax.rem(gj, 2), lax.rem(j, G)]],
                            add=True)

            @pl.when((lax.rem(j, G) == 0) & (j > 0) & (gj + 1 < ngrp))
            def _():
                start_grp(gj + 1, lax.rem(gj + 1, 2))

            t = j + nbuf

            @pl.when(t < nchunk)
            def _():
                tg = t // G

                @pl.when(lax.rem(t, G) == 0)
                def _():
                    wait_grp(lax.rem(tg, 2))

                pltpu.async_copy(
                    scaled_hbm.at[srcg.at[lax.rem(tg, 2), lax.rem(t, G)]],
                    rows.at[rslot], gsems.at[rslot])

            return carry

        lax.fori_loop(0, nchunk, body, 0)
        plsc.subcore_barrier()
        pltpu.sync_copy(acc_sh.at[pl.ds(s * stripe, stripe)],
                        out_hbm.at[c, pl.ds(s * stripe, stripe)])

    return agg_kernel


def _tc_scale(dp3, x, w, npad):
    n, _ = x.shape
    d = w.shape[1]

    def body(dp_ref, x_ref, w_ref, scaled_ref, dinv_ref):
        deg = dp_ref[0] + dp_ref[1] + 1.0
        dinv = lax.rsqrt(deg)
        dinv_n = dinv[:n]
        h = jnp.dot(x_ref[...], w_ref[...], preferred_element_type=jnp.float32)
        scaled_ref[:n] = h * dinv_n
        scaled_ref[n:] = jnp.zeros((npad - n, d), jnp.float32)
        dinv_ref[...] = dinv_n

    return pl.pallas_call(
        body,
        out_shape=[
            jax.ShapeDtypeStruct((npad, d), jnp.float32),
            jax.ShapeDtypeStruct((n, 1), jnp.float32),
        ],
    )(dp3, x, w)


def _tc_finish(partials, scaled, dinv, x, b2, gamma2, beta2):
    n, d = x.shape

    def body(p_ref, scaled_ref, dinv_ref, x_ref, b_ref, g_ref, be_ref, out_ref):
        agg = p_ref[0, :n] + p_ref[1, :n] + scaled_ref[:n]
        conv = agg * dinv_ref[...] + b_ref[...]
        mean = jnp.mean(conv, axis=0, keepdims=True)
        cm = conv - mean
        var = jnp.mean(cm * cm, axis=0, keepdims=True)
        hbn = g_ref[...] * cm * lax.rsqrt(var + EPS) + be_ref[...]
        out_ref[...] = jnp.maximum(jnp.maximum(hbn, 0.0) + x_ref[...], 0.0)

    return pl.pallas_call(
        body,
        out_shape=jax.ShapeDtypeStruct((n, d), jnp.float32),
    )(partials, scaled, dinv, x, b2, gamma2, beta2)


def kernel(x, edge_index, W, b, gamma, beta):
    n, d_in = x.shape
    d = W.shape[1]
    e = edge_index.shape[1]

    npad_deg = (n // 256 + 1) * 256
    npad = (n // 128 + 1) * 128
    stripe = npad // NS
    per_w = -(-e // NW)
    nchunk = -(-(-(-per_w // K)) // 8) * 8
    epad = NW * nchunk * K
    pad_cnt = epad - e

    pad_idx = n + (jnp.arange(pad_cnt, dtype=jnp.int32) % (npad - n))
    src = jnp.concatenate([edge_index[0], pad_idx]).reshape(NW, nchunk, K)
    dst = jnp.concatenate([edge_index[1], pad_idx]).reshape(NW, nchunk, K)

    degp = _sc_degree(NW, nchunk, npad_deg, npad_deg // NS)(dst)
    scaled, dinv = _tc_scale(degp.reshape(NC, npad_deg, 1), x, W, npad)
    partials = _sc_aggregate(NW, nchunk, npad, stripe, d)(scaled, src, dst)
    return _tc_finish(partials, scaled, dinv, x,
                      b.reshape(1, d), gamma.reshape(1, d), beta.reshape(1, d))
